# baseline (speedup 1.0000x reference)
import functools

import jax
import jax.numpy as jnp
from jax import lax
from jax.experimental import pallas as pl
from jax.experimental.pallas import tpu as pltpu
from jax.experimental.pallas import tpu_sc as plsc

_B, _T, _C = 4, 2048, 1024
_G, _V = 2, 320
_VP = 384
_NEG = -1e30
_N = _B * _T
_VD = 256
_TB = 1024
_NBLK_A = 6
_NBLK_B = _N // _TB - _NBLK_A
_NSL_A = _NBLK_A * _TB
_NSL_B = _NBLK_B * _TB

_NC, _NS = 2, 16
_NW = _NC * _NS
_CB = 128


def _tc_body(nblk, last, *refs):
    if last:
        (x_ref, w_ref, ssb_ref, a0, a1, a2, a3, a4, a5,
         idx0_ref, idx1_ref, o0, o1, o2, o3, o4, o5,
         lent_ref, cperp_ref, pperp_ref,
         wsc, ssc, sbsc, probs0, probs1, cnt0, cnt1, cs0, cs1) = refs
        acc_in = (a0, a1, a2, a3, a4, a5)
    else:
        (x_ref, w_ref, ssb_ref,
         idx0_ref, idx1_ref, o0, o1, o2, o3, o4, o5,
         wsc, ssc, sbsc, probs0, probs1, cnt0, cnt1, cs0, cs1) = refs
        acc_in = None
    acc_out = (o0, o1, o2, o3, o4, o5)
    scratch = (probs0, probs1, cnt0, cnt1, cs0, cs1)
    i = pl.program_id(0)

    @pl.when(i == 0)
    def _init():
        if acc_in is None:
            for r in scratch:
                r[...] = jnp.zeros_like(r)
        else:
            for r, src in zip(scratch, acc_in):
                r[...] = src[...]
        wsc[0:_V, :] = w_ref[0:_V, :]
        wsc[_V:_VP, :] = jnp.zeros((_VP - _V, _C), jnp.float32)
        wsc[_VP:_VP + _V, :] = w_ref[_V:2 * _V, :]
        wsc[_VP + _V:2 * _VP, :] = jnp.zeros((_VP - _V, _C), jnp.float32)
        real = lax.broadcasted_iota(jnp.int32, (_G, _VP), 1) < _V
        ssc[...] = jnp.where(real, ssb_ref[0:_G, :], 1.0)
        sbsc[...] = jnp.where(real, ssb_ref[_G:2 * _G, :], _NEG)

    xb = x_ref[...]
    rawp = lax.dot_general(xb, wsc[...], (((1,), (1,)), ((), ())),
                           preferred_element_type=jnp.float32)
    for g, idx_ref, pa, ca, sa in (
            (0, idx0_ref, probs0, cnt0, cs0),
            (1, idx1_ref, probs1, cnt1, cs1)):
        raw = rawp[:, g * _VP:(g + 1) * _VP]
        lg = raw * ssc[g, :][None, :] + sbsc[g, :][None, :]
        m = jnp.max(lg, axis=1, keepdims=True)
        e = jnp.exp(lg - m)
        p = e / jnp.sum(e, axis=1, keepdims=True)
        pa[...] += jnp.sum(p, axis=0, keepdims=True)
        k = jnp.argmax(lg, axis=1).astype(jnp.int32)
        idx_ref[...] = k
        oh = (lax.broadcasted_iota(jnp.int32, (_TB, _VP), 1)
              == k[:, None]).astype(jnp.float32)
        ca[...] += jnp.sum(oh, axis=0, keepdims=True)
        sa[...] += jnp.sum(lg, axis=0, keepdims=True)

    @pl.when(i == nblk - 1)
    def _emit():
        for r, dst in zip(scratch, acc_out):
            dst[...] = r[...]

    if last:
        @pl.when(i == nblk - 1)
        def _finish():
            invn = jnp.float32(1.0 / _N)
            pperp = jnp.float32(0.0)
            cperp = jnp.float32(0.0)
            for pa, ca in ((probs0, cnt0), (probs1, cnt1)):
                ap = pa[...] * invn
                pperp += jnp.exp(-jnp.sum(ap * jnp.log(ap + 1e-7)))
                hp = ca[...] * invn
                cperp += jnp.exp(-jnp.sum(hp * jnp.log(hp + 1e-7)))
            x0 = cs0[...] * invn
            x1 = cs1[...] * invn
            m2 = jnp.maximum(jnp.max(x0), jnp.max(x1))
            e0 = jnp.exp(x0 - m2)
            e1 = jnp.exp(x1 - m2)
            z = jnp.sum(e0) + jnp.sum(e1)
            logz = jnp.log(z)
            lent = (jnp.sum(e0 * ((x0 - m2) - logz))
                    + jnp.sum(e1 * ((x1 - m2) - logz))) / z
            lent_ref[0, 0] = lent
            cperp_ref[0, 0] = cperp
            pperp_ref[0, 0] = pperp


def _tc_stats(x2d, w, ssb, accs, base, nblk, last):
    acc_sds = jax.ShapeDtypeStruct((1, _VP), jnp.float32)
    out_shape = [jax.ShapeDtypeStruct((nblk * _TB,), jnp.int32)] * 2 \
        + [acc_sds] * 6
    out_specs = [pl.BlockSpec((_TB,), lambda i: (i,))] * 2 + \
                [pl.BlockSpec((1, _VP), lambda i: (0, 0))] * 6
    in_specs = [
        pl.BlockSpec((_TB, _C), lambda i: (i + base, 0)),
        pl.BlockSpec((_G * _V, _C), lambda i: (0, 0)),
        pl.BlockSpec((2 * _G, _VP), lambda i: (0, 0)),
    ]
    if last:
        out_shape += [jax.ShapeDtypeStruct((1, 1), jnp.float32)] * 3
        out_specs += [pl.BlockSpec(memory_space=pltpu.SMEM)] * 3
        in_specs += [pl.BlockSpec((1, _VP), lambda i: (0, 0))] * 6
    return pl.pallas_call(
        functools.partial(_tc_body, nblk, last),
        grid=(nblk,),
        in_specs=in_specs,
        out_specs=out_specs,
        out_shape=out_shape,
        scratch_shapes=[pltpu.VMEM((_G * _VP, _C), jnp.float32),
                        pltpu.VMEM((_G, _VP), jnp.float32),
                        pltpu.VMEM((_G, _VP), jnp.float32)]
        + [pltpu.VMEM((1, _VP), jnp.float32) for _ in range(6)],
        compiler_params=pltpu.CompilerParams(
            dimension_semantics=("arbitrary",)),
    )(x2d, w, ssb, *accs)


def _sc_body(base_tok, tpw, chunks, tableg_hbm, k0_hbm, k1_hbm, out_hbm,
             kv, buf0, buf1, gs0, gs1, ss0, ss1):
    wid = lax.axis_index("s") * _NC + lax.axis_index("c")
    g = wid & 1
    blk = wid >> 1
    col0 = g * _VD
    tok0 = base_tok + blk * tpw

    @pl.when(g == 0)
    def _load0():
        pltpu.sync_copy(k0_hbm.at[pl.ds(blk * tpw, tpw)], kv)

    @pl.when(g == 1)
    def _load1():
        pltpu.sync_copy(k1_hbm.at[pl.ds(blk * tpw, tpw)], kv)

    tab = tableg_hbm.at[g]
    bufs = (buf0, buf1)
    gsems = (gs0, gs1)
    ssems = (ss0, ss1)
    gh = [None, None]
    sh = [None, None]
    gh[0] = pltpu.async_copy(tab.at[kv.at[pl.ds(0, _CB)]], buf0, gs0)
    for c in range(chunks):
        n = c + 1
        if n < chunks:
            if sh[n % 2] is not None:
                sh[n % 2].wait()
            gh[n % 2] = pltpu.async_copy(
                tab.at[kv.at[pl.ds(n * _CB, _CB)]], bufs[n % 2],
                gsems[n % 2])
        gh[c % 2].wait()
        sh[c % 2] = pltpu.async_copy(
            bufs[c % 2],
            out_hbm.at[pl.ds(tok0 + c * _CB, _CB), pl.ds(col0, _VD)],
            ssems[c % 2])
    for c in range(max(0, chunks - 2), chunks):
        sh[c % 2].wait()


def _sc_gather(tableg, k0, k1, qref, base_tok, tpw):
    chunks = tpw // _CB
    mesh = plsc.VectorSubcoreMesh(core_axis_name="c", subcore_axis_name="s")
    out_type = () if qref is not None else jax.ShapeDtypeStruct(
        (_N, _G * _VD), jnp.float32)
    run = functools.partial(
        pl.kernel,
        mesh=mesh,
        out_type=out_type,
        scratch_types=[
            pltpu.VMEM((tpw,), jnp.int32),
            pltpu.VMEM((_CB, _VD), jnp.float32),
            pltpu.VMEM((_CB, _VD), jnp.float32),
            pltpu.SemaphoreType.DMA,
            pltpu.SemaphoreType.DMA,
            pltpu.SemaphoreType.DMA,
            pltpu.SemaphoreType.DMA,
        ],
    )(functools.partial(_sc_body, base_tok, tpw, chunks))
    if qref is None:
        return run(tableg, k0, k1)
    run(tableg, k0, k1, qref)


def kernel(x, W, b, vars_p, scaling):
    avg = scaling.mean()
    scale = 1.0 + 10.0 * (scaling - avg)
    ssb = jnp.pad(jnp.concatenate(
        [scale.reshape(_G, _V), (b * scale).reshape(_G, _V)]),
        ((0, 0), (0, _VP - _V)))

    x2d = x.reshape(_N, _C)
    tableg = vars_p.reshape(_G, _V, _VD)

    k0a, k1a, *accs_a = _tc_stats(x2d, W, ssb, [], 0, _NBLK_A, False)
    out_a = _sc_gather(tableg, k0a, k1a, None, 0, _NSL_A // (_NW // 2))
    qref = jax.new_ref(out_a)

    out_b = _tc_stats(x2d, W, ssb, accs_a, _NBLK_A, _NBLK_B, True)
    k0b, k1b = out_b[0], out_b[1]
    lent, cperp, pperp = out_b[8], out_b[9], out_b[10]
    _sc_gather(tableg, k0b, k1b, qref, _NSL_A, _NSL_B // (_NW // 2))

    q = qref[...].reshape(_B, _T, _G * _VD)
    return (q, lent.reshape(()), cperp.reshape(()), pperp.reshape(()))


# speedup vs baseline: 1.0171x; 1.0171x over previous
import functools

import jax
import jax.numpy as jnp
from jax import lax
from jax.experimental import pallas as pl
from jax.experimental.pallas import tpu as pltpu
from jax.experimental.pallas import tpu_sc as plsc

_B, _T, _C = 4, 2048, 1024
_G, _V = 2, 320
_VP = 384
_NEG = -1e30
_N = _B * _T
_VD = 256
_TB = 1024
_NBLK_A = 6
_NBLK_B = _N // _TB - _NBLK_A
_NSL_A = _NBLK_A * _TB
_NSL_B = _NBLK_B * _TB

_NC, _NS = 2, 16
_NW = _NC * _NS
_CB = 128


def _tc_body(nblk, last, *refs):
    if last:
        (x_ref, w_ref, ssb_ref, a0, a1, a2, a3, a4, a5,
         idx0_ref, idx1_ref, o0, o1, o2, o3, o4, o5,
         lent_ref, cperp_ref, pperp_ref,
         wsc, ssc, sbsc, probs0, probs1, cnt0, cnt1, cs0, cs1) = refs
        acc_in = (a0, a1, a2, a3, a4, a5)
    else:
        (x_ref, w_ref, ssb_ref,
         idx0_ref, idx1_ref, o0, o1, o2, o3, o4, o5,
         wsc, ssc, sbsc, probs0, probs1, cnt0, cnt1, cs0, cs1) = refs
        acc_in = None
    acc_out = (o0, o1, o2, o3, o4, o5)
    scratch = (probs0, probs1, cnt0, cnt1, cs0, cs1)
    i = pl.program_id(0)

    @pl.when(i == 0)
    def _init():
        if acc_in is None:
            for r in scratch:
                r[...] = jnp.zeros_like(r)
        else:
            for r, src in zip(scratch, acc_in):
                r[...] = src[...]
        wsc[0:_V, :] = w_ref[0:_V, :]
        wsc[_V:_VP, :] = jnp.zeros((_VP - _V, _C), jnp.float32)
        wsc[_VP:_VP + _V, :] = w_ref[_V:2 * _V, :]
        wsc[_VP + _V:2 * _VP, :] = jnp.zeros((_VP - _V, _C), jnp.float32)
        real = lax.broadcasted_iota(jnp.int32, (_G, _VP), 1) < _V
        scv = jnp.where(real, ssb_ref[0:_G, :], 0.0)
        bv = ssb_ref[_G:2 * _G, :]
        avg = jnp.sum(scv) * jnp.float32(1.0 / (_G * _V))
        scale = 1.0 + 10.0 * (scv - avg)
        ssc[...] = jnp.where(real, scale, 1.0)
        sbsc[...] = jnp.where(real, bv * scale, _NEG)

    xb = x_ref[...]
    rawp = lax.dot_general(xb, wsc[...], (((1,), (1,)), ((), ())),
                           preferred_element_type=jnp.float32)
    for g, idx_ref, pa, ca, sa in (
            (0, idx0_ref, probs0, cnt0, cs0),
            (1, idx1_ref, probs1, cnt1, cs1)):
        raw = rawp[:, g * _VP:(g + 1) * _VP]
        lg = raw * ssc[g, :][None, :] + sbsc[g, :][None, :]
        m = jnp.max(lg, axis=1, keepdims=True)
        e = jnp.exp(lg - m)
        p = e / jnp.sum(e, axis=1, keepdims=True)
        pa[...] += jnp.sum(p, axis=0, keepdims=True)
        k = jnp.argmax(lg, axis=1).astype(jnp.int32)
        idx_ref[...] = k
        oh = (lax.broadcasted_iota(jnp.int32, (_TB, _VP), 1)
              == k[:, None]).astype(jnp.float32)
        ca[...] += jnp.sum(oh, axis=0, keepdims=True)
        sa[...] += jnp.sum(lg, axis=0, keepdims=True)

    @pl.when(i == nblk - 1)
    def _emit():
        for r, dst in zip(scratch, acc_out):
            dst[...] = r[...]

    if last:
        @pl.when(i == nblk - 1)
        def _finish():
            invn = jnp.float32(1.0 / _N)
            pperp = jnp.float32(0.0)
            cperp = jnp.float32(0.0)
            for pa, ca in ((probs0, cnt0), (probs1, cnt1)):
                ap = pa[...] * invn
                pperp += jnp.exp(-jnp.sum(ap * jnp.log(ap + 1e-7)))
                hp = ca[...] * invn
                cperp += jnp.exp(-jnp.sum(hp * jnp.log(hp + 1e-7)))
            x0 = cs0[...] * invn
            x1 = cs1[...] * invn
            m2 = jnp.maximum(jnp.max(x0), jnp.max(x1))
            e0 = jnp.exp(x0 - m2)
            e1 = jnp.exp(x1 - m2)
            z = jnp.sum(e0) + jnp.sum(e1)
            logz = jnp.log(z)
            lent = (jnp.sum(e0 * ((x0 - m2) - logz))
                    + jnp.sum(e1 * ((x1 - m2) - logz))) / z
            lent_ref[0, 0] = lent
            cperp_ref[0, 0] = cperp
            pperp_ref[0, 0] = pperp


def _tc_stats(x2d, w, ssb, accs, base, nblk, last):
    acc_sds = jax.ShapeDtypeStruct((1, _VP), jnp.float32)
    out_shape = [jax.ShapeDtypeStruct((nblk * _TB,), jnp.int32)] * 2 \
        + [acc_sds] * 6
    out_specs = [pl.BlockSpec((_TB,), lambda i: (i,))] * 2 + \
                [pl.BlockSpec((1, _VP), lambda i: (0, 0))] * 6
    in_specs = [
        pl.BlockSpec((_TB, _C), lambda i: (i + base, 0)),
        pl.BlockSpec((_G * _V, _C), lambda i: (0, 0)),
        pl.BlockSpec((2 * _G, _VP), lambda i: (0, 0)),

    ]
    if last:
        out_shape += [jax.ShapeDtypeStruct((1, 1), jnp.float32)] * 3
        out_specs += [pl.BlockSpec(memory_space=pltpu.SMEM)] * 3
        in_specs += [pl.BlockSpec((1, _VP), lambda i: (0, 0))] * 6
    return pl.pallas_call(
        functools.partial(_tc_body, nblk, last),
        grid=(nblk,),
        in_specs=in_specs,
        out_specs=out_specs,
        out_shape=out_shape,
        scratch_shapes=[pltpu.VMEM((_G * _VP, _C), jnp.float32),
                        pltpu.VMEM((_G, _VP), jnp.float32),
                        pltpu.VMEM((_G, _VP), jnp.float32)]
        + [pltpu.VMEM((1, _VP), jnp.float32) for _ in range(6)],
        compiler_params=pltpu.CompilerParams(
            dimension_semantics=("arbitrary",)),
    )(x2d, w, ssb, *accs)


def _sc_body(base_tok, tpw, chunks, tableg_hbm, k0_hbm, k1_hbm, out_hbm,
             kv, buf0, buf1, gs0, gs1, ss0, ss1):
    wid = lax.axis_index("s") * _NC + lax.axis_index("c")
    g = wid & 1
    blk = wid >> 1
    col0 = g * _VD
    tok0 = base_tok + blk * tpw

    @pl.when(g == 0)
    def _load0():
        pltpu.sync_copy(k0_hbm.at[pl.ds(blk * tpw, tpw)], kv)

    @pl.when(g == 1)
    def _load1():
        pltpu.sync_copy(k1_hbm.at[pl.ds(blk * tpw, tpw)], kv)

    tab = tableg_hbm.at[g]
    bufs = (buf0, buf1)
    gsems = (gs0, gs1)
    ssems = (ss0, ss1)
    gh = [None, None]
    sh = [None, None]
    gh[0] = pltpu.async_copy(tab.at[kv.at[pl.ds(0, _CB)]], buf0, gs0)
    for c in range(chunks):
        n = c + 1
        if n < chunks:
            if sh[n % 2] is not None:
                sh[n % 2].wait()
            gh[n % 2] = pltpu.async_copy(
                tab.at[kv.at[pl.ds(n * _CB, _CB)]], bufs[n % 2],
                gsems[n % 2])
        gh[c % 2].wait()
        sh[c % 2] = pltpu.async_copy(
            bufs[c % 2],
            out_hbm.at[pl.ds(tok0 + c * _CB, _CB), pl.ds(col0, _VD)],
            ssems[c % 2])
    for c in range(max(0, chunks - 2), chunks):
        sh[c % 2].wait()


def _sc_gather(tableg, k0, k1, qref, base_tok, tpw):
    chunks = tpw // _CB
    mesh = plsc.VectorSubcoreMesh(core_axis_name="c", subcore_axis_name="s")
    out_type = () if qref is not None else jax.ShapeDtypeStruct(
        (_N, _G * _VD), jnp.float32)
    run = functools.partial(
        pl.kernel,
        mesh=mesh,
        out_type=out_type,
        scratch_types=[
            pltpu.VMEM((tpw,), jnp.int32),
            pltpu.VMEM((_CB, _VD), jnp.float32),
            pltpu.VMEM((_CB, _VD), jnp.float32),
            pltpu.SemaphoreType.DMA,
            pltpu.SemaphoreType.DMA,
            pltpu.SemaphoreType.DMA,
            pltpu.SemaphoreType.DMA,
        ],
    )(functools.partial(_sc_body, base_tok, tpw, chunks))
    if qref is None:
        return run(tableg, k0, k1)
    run(tableg, k0, k1, qref)


def kernel(x, W, b, vars_p, scaling):
    ssb = jnp.concatenate(
        [scaling.reshape(_G, _V), b.reshape(_G, _V)])

    x2d = x.reshape(_N, _C)
    tableg = vars_p.reshape(_G, _V, _VD)

    k0a, k1a, *accs_a = _tc_stats(x2d, W, ssb, [], 0, _NBLK_A, False)
    out_a = _sc_gather(tableg, k0a, k1a, None, 0, _NSL_A // (_NW // 2))
    qref = jax.new_ref(out_a)

    out_b = _tc_stats(x2d, W, ssb, accs_a, _NBLK_A, _NBLK_B, True)
    k0b, k1b = out_b[0], out_b[1]
    lent, cperp, pperp = out_b[8], out_b[9], out_b[10]
    _sc_gather(tableg, k0b, k1b, qref, _NSL_A, _NSL_B // (_NW // 2))

    q = qref[...].reshape(_B, _T, _G * _VD)
    return (q, lent.reshape(()), cperp.reshape(()), pperp.reshape(()))
